# baseline (speedup 1.0000x reference)
import functools

import jax
import jax.numpy as jnp
from jax import lax
from jax.experimental import pallas as pl
from jax.experimental.pallas import tpu as pltpu
from jax.experimental.pallas import tpu_sc as plsc

E = 8
TOP_K = 2
D = 1024
F = 1024
T = 2048
ALPHA = 1.702
LIMIT = 7.0

BLK = 256
NB = T * TOP_K // BLK + E
S_PAD = NB * BLK

NC = 2
NS = 16
NW = NC * NS
LANES = 16

def _make_gather_body(rows_per_w, gch):
    nch = rows_per_w // gch

    def body(x_hbm, idx_hbm, out_hbm, idx_v, rows0_v, rows1_v,
             sem_g0, sem_g1, sem_w):
        wid = lax.axis_index("s") * NC + lax.axis_index("c")
        base = wid * rows_per_w
        bufs = (rows0_v, rows1_v)
        gsems = (sem_g0, sem_g1)
        pltpu.sync_copy(idx_hbm.at[pl.ds(base, rows_per_w)], idx_v)
        gathers = [None] * nch
        writes = [None] * nch
        gathers[0] = pltpu.async_copy(
            x_hbm.at[idx_v.at[pl.ds(0, gch)]], bufs[0], gsems[0])
        for c in range(nch):
            if c + 1 < nch:
                if c >= 1:
                    writes[c - 1].wait()
                gathers[c + 1] = pltpu.async_copy(
                    x_hbm.at[idx_v.at[pl.ds((c + 1) * gch, gch)]],
                    bufs[(c + 1) % 2], gsems[(c + 1) % 2])
            gathers[c].wait()
            writes[c] = pltpu.async_copy(
                bufs[c % 2], out_hbm.at[pl.ds(base + c * gch, gch)], sem_w)
        writes[nch - 1].wait()
        if nch >= 2:
            writes[nch - 2].wait()

    return body


GCH_X = 48
GCH_P = 32


@functools.lru_cache(maxsize=None)
def _sc_kernels():
    mesh = plsc.VectorSubcoreMesh(core_axis_name="c", subcore_axis_name="s")

    def make(total_rows, gch):
        rows_per_w = total_rows // NW
        return pl.kernel(
            _make_gather_body(rows_per_w, gch),
            out_type=jax.ShapeDtypeStruct((total_rows, D), jnp.float32),
            mesh=mesh,
            scratch_types=[
                pltpu.VMEM((rows_per_w,), jnp.int32),
                pltpu.VMEM((gch, D), jnp.float32),
                pltpu.VMEM((gch, D), jnp.float32),
                pltpu.SemaphoreType.DMA,
                pltpu.SemaphoreType.DMA,
                pltpu.SemaphoreType.DMA,
            ],
        )

    return make(S_PAD, GCH_X), make(2 * T, GCH_P)


def _ffn_body(meta_ref, x_ref, w1_ref, w2x_ref, b1_ref, b2_ref, w_ref,
              o_ref):
    i = pl.program_id(0)

    @pl.when(meta_ref[NB + i] == 1)
    def _():
        x = x_ref[...]
        gu = jnp.dot(x, w1_ref[0], preferred_element_type=jnp.float32)
        gu = gu + b1_ref[0]
        g = jnp.minimum(gu, LIMIT)
        u = pltpu.roll(gu, 2 * F - 1, 1)
        u = jnp.clip(u, -LIMIT, LIMIT)
        glu = g * jax.nn.sigmoid(g * ALPHA)
        gated = (u + 1.0) * glu
        o = jnp.dot(gated, w2x_ref[0], preferred_element_type=jnp.float32)
        o_ref[...] = (o + b2_ref[0]) * w_ref[0]


def _grouped_ffn(meta, x_sorted, w1, w2x, b1, b2, w_rows):
    grid_spec = pltpu.PrefetchScalarGridSpec(
        num_scalar_prefetch=1,
        grid=(NB,),
        in_specs=[
            pl.BlockSpec((BLK, D), lambda i, m: (i, 0)),
            pl.BlockSpec((1, D, 2 * F), lambda i, m: (m[i], 0, 0)),
            pl.BlockSpec((1, 2 * F, D), lambda i, m: (m[i], 0, 0)),
            pl.BlockSpec((1, 1, 2 * F), lambda i, m: (m[i], 0, 0)),
            pl.BlockSpec((1, 1, D), lambda i, m: (m[i], 0, 0)),
            pl.BlockSpec((1, BLK, 1), lambda i, m: (i, 0, 0)),
        ],
        out_specs=pl.BlockSpec((BLK, D), lambda i, m: (i, 0)),
    )
    return pl.pallas_call(
        _ffn_body,
        grid_spec=grid_spec,
        out_shape=jax.ShapeDtypeStruct((S_PAD, D), jnp.float32),
    )(meta, x_sorted, w1, w2x, b1, b2, w_rows)


def _copy_body(x_ref, o_ref):
    o_ref[...] = x_ref[...]


def _tc_copy(x):
    return pl.pallas_call(
        _copy_body,
        grid=(T // 512,),
        in_specs=[pl.BlockSpec((512, D), lambda i: (i, 0))],
        out_specs=pl.BlockSpec((512, D), lambda i: (i, 0)),
        out_shape=jax.ShapeDtypeStruct((T, D), jnp.float32),
    )(x)


ADD_BLK = 512


def _add_body(a_ref, b_ref, o_ref):
    o_ref[...] = a_ref[...] + b_ref[...]


def _pair_add(ab):
    return pl.pallas_call(
        _add_body,
        grid=(T // ADD_BLK,),
        in_specs=[
            pl.BlockSpec((ADD_BLK, D), lambda i: (i, 0)),
            pl.BlockSpec((ADD_BLK, D), lambda i: (i + T // ADD_BLK, 0)),
        ],
        out_specs=pl.BlockSpec((ADD_BLK, D), lambda i: (i, 0)),
        out_shape=jax.ShapeDtypeStruct((T, D), jnp.float32),
    )(ab, ab)


def kernel(hidden_states, router_weight, router_bias, gate_up_proj,
           gate_up_proj_bias, down_proj, down_proj_bias):
    batch = hidden_states.shape[0]
    x = hidden_states.reshape(T, D)

    logits = x @ router_weight.T + router_bias
    top_vals, top_idx = jax.lax.top_k(logits, TOP_K)
    rw = jax.nn.softmax(top_vals, axis=-1)

    e_flat = top_idx.reshape(-1).astype(jnp.int32)
    onehot = (e_flat[:, None] == jnp.arange(E, dtype=jnp.int32)[None, :])
    csum = jnp.cumsum(onehot.astype(jnp.int32), axis=0)
    rank = jnp.take_along_axis(csum, e_flat[:, None], axis=1)[:, 0] - 1
    counts = csum[-1]
    nblk = (counts + BLK - 1) // BLK
    blk_end = jnp.cumsum(nblk)
    blk_start = blk_end - nblk
    slot = blk_start[e_flat] * BLK + rank

    total_blk = blk_end[-1]
    bids = jnp.arange(NB, dtype=jnp.int32)
    bexp = jnp.sum(bids[:, None] >= blk_end[None, :], axis=1).astype(jnp.int32)
    last_e = jnp.max(jnp.where(counts > 0, jnp.arange(E, dtype=jnp.int32), -1))
    active = (bids < total_blk).astype(jnp.int32)
    bexp = jnp.where(active == 1, bexp, last_e)
    meta = jnp.concatenate([bexp, active])

    order = jnp.argsort(e_flat, stable=True).astype(jnp.int32)
    count_off = jnp.cumsum(counts) - counts
    sids = jnp.arange(S_PAD, dtype=jnp.int32)
    e_s = bexp[sids // BLK]
    r_s = sids - blk_start[e_s] * BLK
    valid_s = r_s < counts[e_s]
    p_s = jnp.clip(count_off[e_s] + r_s, 0, 2 * T - 1)
    src_token = jnp.where(valid_s, order[p_s] // TOP_K, 0).astype(jnp.int32)
    w_flat = rw.reshape(-1)
    w_rows = jnp.where(valid_s, w_flat[order[p_s]], 0.0)
    w_rows = w_rows.reshape(NB, BLK, 1).astype(jnp.float32)
    pair_slots = jnp.concatenate([slot[0::2], slot[1::2]])

    w2x = jnp.concatenate(
        [down_proj[:, :, None, :],
         jnp.zeros((E, F, 1, D), down_proj.dtype)], axis=2
    ).reshape(E, 2 * F, D)
    b1 = gate_up_proj_bias.reshape(E, 1, 2 * F)
    b2 = down_proj_bias.reshape(E, 1, D)

    sc_gather, sc_pair_gather = _sc_kernels()
    x_sorted = sc_gather(_tc_copy(x), src_token)
    out_sorted = _grouped_ffn(meta, x_sorted, gate_up_proj, w2x, b1, b2,
                              w_rows)
    ab = sc_pair_gather(out_sorted, pair_slots)
    out = _pair_add(ab)
    return out.reshape(batch, T, D)


# speedup vs baseline: 1.2658x; 1.2658x over previous
import functools

import jax
import jax.numpy as jnp
from jax import lax
from jax.experimental import pallas as pl
from jax.experimental.pallas import tpu as pltpu
from jax.experimental.pallas import tpu_sc as plsc

E = 8
TOP_K = 2
D = 1024
F = 1024
T = 2048
ALPHA = 1.702
LIMIT = 7.0

BLK = 256
NB = T * TOP_K // BLK + E
S_PAD = NB * BLK

NC = 2
NS = 16
NW = NC * NS
LANES = 16

def _make_gather_body(rows_per_w, gch):
    nch = rows_per_w // gch

    def body(x_hbm, idx_hbm, out_hbm, idx_v, rows0_v, rows1_v,
             sem_g0, sem_g1, sem_w):
        wid = lax.axis_index("s") * NC + lax.axis_index("c")
        base = wid * rows_per_w
        bufs = (rows0_v, rows1_v)
        gsems = (sem_g0, sem_g1)
        pltpu.sync_copy(idx_hbm.at[pl.ds(base, rows_per_w)], idx_v)
        gathers = [None] * nch
        writes = [None] * nch
        gathers[0] = pltpu.async_copy(
            x_hbm.at[idx_v.at[pl.ds(0, gch)]], bufs[0], gsems[0])
        for c in range(nch):
            if c + 1 < nch:
                if c >= 1:
                    writes[c - 1].wait()
                gathers[c + 1] = pltpu.async_copy(
                    x_hbm.at[idx_v.at[pl.ds((c + 1) * gch, gch)]],
                    bufs[(c + 1) % 2], gsems[(c + 1) % 2])
            gathers[c].wait()
            writes[c] = pltpu.async_copy(
                bufs[c % 2], out_hbm.at[pl.ds(base + c * gch, gch)], sem_w)
        writes[nch - 1].wait()
        if nch >= 2:
            writes[nch - 2].wait()

    return body


GCH_X = 32
GCH_P = 32


@functools.lru_cache(maxsize=None)
def _sc_kernels():
    mesh = plsc.VectorSubcoreMesh(core_axis_name="c", subcore_axis_name="s")

    def make(total_rows, gch):
        rows_per_w = total_rows // NW
        return pl.kernel(
            _make_gather_body(rows_per_w, gch),
            out_type=jax.ShapeDtypeStruct((total_rows, D), jnp.float32),
            mesh=mesh,
            scratch_types=[
                pltpu.VMEM((rows_per_w,), jnp.int32),
                pltpu.VMEM((gch, D), jnp.float32),
                pltpu.VMEM((gch, D), jnp.float32),
                pltpu.SemaphoreType.DMA,
                pltpu.SemaphoreType.DMA,
                pltpu.SemaphoreType.DMA,
            ],
        )

    return make(S_PAD, GCH_X), make(2 * T, GCH_P)


def _ffn_body(meta_ref, x_ref, w1_ref, w2x_ref, b1_ref, b2_ref, w_ref,
              o_ref):
    i = pl.program_id(0)

    @pl.when(meta_ref[NB + i] == 1)
    def _():
        x = x_ref[...]
        gu = jnp.dot(x, w1_ref[0], preferred_element_type=jnp.float32)
        gu = gu + b1_ref[0]
        g = jnp.minimum(gu, LIMIT)
        u = pltpu.roll(gu, 2 * F - 1, 1)
        u = jnp.clip(u, -LIMIT, LIMIT)
        glu = g * jax.nn.sigmoid(g * ALPHA)
        gated = (u + 1.0) * glu
        o = jnp.dot(gated, w2x_ref[0], preferred_element_type=jnp.float32)
        o_ref[...] = (o + b2_ref[0]) * w_ref[0]


def _grouped_ffn(meta, x_sorted, w1, w2x, b1, b2, w_rows):
    grid_spec = pltpu.PrefetchScalarGridSpec(
        num_scalar_prefetch=1,
        grid=(NB,),
        in_specs=[
            pl.BlockSpec((BLK, D), lambda i, m: (i, 0)),
            pl.BlockSpec((1, D, 2 * F), lambda i, m: (m[i], 0, 0)),
            pl.BlockSpec((1, 2 * F, D), lambda i, m: (m[i], 0, 0)),
            pl.BlockSpec((1, 1, 2 * F), lambda i, m: (m[i], 0, 0)),
            pl.BlockSpec((1, 1, D), lambda i, m: (m[i], 0, 0)),
            pl.BlockSpec((1, BLK, 1), lambda i, m: (i, 0, 0)),
        ],
        out_specs=pl.BlockSpec((BLK, D), lambda i, m: (i, 0)),
    )
    return pl.pallas_call(
        _ffn_body,
        grid_spec=grid_spec,
        out_shape=jax.ShapeDtypeStruct((S_PAD, D), jnp.float32),
    )(meta, x_sorted, w1, w2x, b1, b2, w_rows)


def _w2x_body(w2_ref, o_ref):
    w = w2_ref[0]
    z = jnp.zeros_like(w)
    o_ref[...] = jnp.stack([w, z], axis=1).reshape(1, 2 * F, D)


def _tc_w2x(down_proj):
    return pl.pallas_call(
        _w2x_body,
        grid=(E,),
        in_specs=[pl.BlockSpec((1, F, D), lambda i: (i, 0, 0))],
        out_specs=pl.BlockSpec((1, 2 * F, D), lambda i: (i, 0, 0)),
        out_shape=jax.ShapeDtypeStruct((E, 2 * F, D), jnp.float32),
    )(down_proj)


ADD_BLK = 512


def _add_body(a_ref, b_ref, o_ref):
    o_ref[...] = a_ref[...] + b_ref[...]


def _pair_add(ab):
    return pl.pallas_call(
        _add_body,
        grid=(T // ADD_BLK,),
        in_specs=[
            pl.BlockSpec((ADD_BLK, D), lambda i: (i, 0)),
            pl.BlockSpec((ADD_BLK, D), lambda i: (i + T // ADD_BLK, 0)),
        ],
        out_specs=pl.BlockSpec((ADD_BLK, D), lambda i: (i, 0)),
        out_shape=jax.ShapeDtypeStruct((T, D), jnp.float32),
    )(ab, ab)


def kernel(hidden_states, router_weight, router_bias, gate_up_proj,
           gate_up_proj_bias, down_proj, down_proj_bias):
    batch = hidden_states.shape[0]
    x = hidden_states.reshape(T, D)

    logits = x @ router_weight.T + router_bias
    top_vals, top_idx = jax.lax.top_k(logits, TOP_K)
    rw = jax.nn.softmax(top_vals, axis=-1)

    e_flat = top_idx.reshape(-1).astype(jnp.int32)
    onehot = (e_flat[:, None] == jnp.arange(E, dtype=jnp.int32)[None, :])
    csum = jnp.cumsum(onehot.astype(jnp.int32), axis=0)
    rank = jnp.take_along_axis(csum, e_flat[:, None], axis=1)[:, 0] - 1
    counts = csum[-1]
    nblk = (counts + BLK - 1) // BLK
    blk_end = jnp.cumsum(nblk)
    blk_start = blk_end - nblk
    slot = blk_start[e_flat] * BLK + rank

    total_blk = blk_end[-1]
    bids = jnp.arange(NB, dtype=jnp.int32)
    bexp = jnp.sum(bids[:, None] >= blk_end[None, :], axis=1).astype(jnp.int32)
    last_e = jnp.max(jnp.where(counts > 0, jnp.arange(E, dtype=jnp.int32), -1))
    active = (bids < total_blk).astype(jnp.int32)
    bexp = jnp.where(active == 1, bexp, last_e)
    meta = jnp.concatenate([bexp, active])

    order = jnp.argsort(e_flat, stable=True).astype(jnp.int32)
    count_off = jnp.cumsum(counts) - counts
    sids = jnp.arange(S_PAD, dtype=jnp.int32)
    e_s = bexp[sids // BLK]
    r_s = sids - blk_start[e_s] * BLK
    valid_s = r_s < counts[e_s]
    p_s = jnp.clip(count_off[e_s] + r_s, 0, 2 * T - 1)
    src_token = jnp.where(valid_s, order[p_s] // TOP_K, 0).astype(jnp.int32)
    w_flat = rw.reshape(-1)
    w_rows = jnp.where(valid_s, w_flat[order[p_s]], 0.0)
    w_rows = w_rows.reshape(NB, BLK, 1).astype(jnp.float32)
    pair_slots = jnp.concatenate([slot[0::2], slot[1::2]])

    w2x = _tc_w2x(down_proj)
    b1 = gate_up_proj_bias.reshape(E, 1, 2 * F)
    b2 = down_proj_bias.reshape(E, 1, D)

    sc_gather, sc_pair_gather = _sc_kernels()
    x_sorted = sc_gather(x, src_token)
    out_sorted = _grouped_ffn(meta, x_sorted, gate_up_proj, w2x, b1, b2,
                              w_rows)
    ab = sc_pair_gather(out_sorted, pair_slots)
    out = _pair_add(ab)
    return out.reshape(batch, T, D)


# speedup vs baseline: 1.6292x; 1.2871x over previous
import functools

import jax
import jax.numpy as jnp
from jax import lax
from jax.experimental import pallas as pl
from jax.experimental.pallas import tpu as pltpu
from jax.experimental.pallas import tpu_sc as plsc

E = 8
TOP_K = 2
D = 1024
F = 1024
T = 2048
ALPHA = 1.702
LIMIT = 7.0

BLK = 256
NB = T * TOP_K // BLK + E
S_PAD = NB * BLK

NC = 2
NS = 16
NW = NC * NS
LANES = 16

def _make_gather_body(rows_per_w, gch):
    nch = rows_per_w // gch

    def body(x_hbm, idx_hbm, out_hbm, idx_v, rows0_v, rows1_v,
             sem_g0, sem_g1, sem_w):
        wid = lax.axis_index("s") * NC + lax.axis_index("c")
        base = wid * rows_per_w
        bufs = (rows0_v, rows1_v)
        gsems = (sem_g0, sem_g1)
        pltpu.sync_copy(idx_hbm.at[pl.ds(base, rows_per_w)], idx_v)
        gathers = [None] * nch
        writes = [None] * nch
        gathers[0] = pltpu.async_copy(
            x_hbm.at[idx_v.at[pl.ds(0, gch)]], bufs[0], gsems[0])
        for c in range(nch):
            if c + 1 < nch:
                if c >= 1:
                    writes[c - 1].wait()
                gathers[c + 1] = pltpu.async_copy(
                    x_hbm.at[idx_v.at[pl.ds((c + 1) * gch, gch)]],
                    bufs[(c + 1) % 2], gsems[(c + 1) % 2])
            gathers[c].wait()
            writes[c] = pltpu.async_copy(
                bufs[c % 2], out_hbm.at[pl.ds(base + c * gch, gch)], sem_w)
        writes[nch - 1].wait()
        if nch >= 2:
            writes[nch - 2].wait()

    return body


GCH_X = 32
GCH_P = 32


@functools.lru_cache(maxsize=None)
def _sc_kernels():
    mesh = plsc.VectorSubcoreMesh(core_axis_name="c", subcore_axis_name="s")

    def make(total_rows, gch):
        rows_per_w = total_rows // NW
        return pl.kernel(
            _make_gather_body(rows_per_w, gch),
            out_type=jax.ShapeDtypeStruct((total_rows, D), jnp.float32),
            mesh=mesh,
            scratch_types=[
                pltpu.VMEM((rows_per_w,), jnp.int32),
                pltpu.VMEM((gch, D), jnp.float32),
                pltpu.VMEM((gch, D), jnp.float32),
                pltpu.SemaphoreType.DMA,
                pltpu.SemaphoreType.DMA,
                pltpu.SemaphoreType.DMA,
            ],
        )

    return make(S_PAD, GCH_X), make(2 * T, GCH_P)


def _ffn_body(meta_ref, x_ref, w1_ref, w2x_ref, b1_ref, b2_ref, w_ref,
              o_ref):
    i = pl.program_id(0)

    @pl.when(meta_ref[NB + i] == 1)
    def _():
        x = x_ref[...]
        gu = jnp.dot(x, w1_ref[0], preferred_element_type=jnp.float32)
        gu = gu + b1_ref[0]
        g = jnp.minimum(gu, LIMIT)
        u = pltpu.roll(gu, 2 * F - 1, 1)
        u = jnp.clip(u, -LIMIT, LIMIT)
        glu = g * jax.nn.sigmoid(g * ALPHA)
        gated = (u + 1.0) * glu
        o = jnp.dot(gated, w2x_ref[0], preferred_element_type=jnp.float32)
        o_ref[...] = (o + b2_ref[0]) * w_ref[0]


def _grouped_ffn(meta, x_sorted, w1, w2x, b1, b2, w_rows):
    grid_spec = pltpu.PrefetchScalarGridSpec(
        num_scalar_prefetch=1,
        grid=(NB,),
        in_specs=[
            pl.BlockSpec((BLK, D), lambda i, m: (i, 0)),
            pl.BlockSpec((1, D, 2 * F), lambda i, m: (m[i], 0, 0)),
            pl.BlockSpec((1, 2 * F, D), lambda i, m: (m[i], 0, 0)),
            pl.BlockSpec((1, 1, 2 * F), lambda i, m: (m[i], 0, 0)),
            pl.BlockSpec((1, 1, D), lambda i, m: (m[i], 0, 0)),
            pl.BlockSpec((1, BLK, 1), lambda i, m: (i, 0, 0)),
        ],
        out_specs=pl.BlockSpec((BLK, D), lambda i, m: (i, 0)),
    )
    return pl.pallas_call(
        _ffn_body,
        grid_spec=grid_spec,
        out_shape=jax.ShapeDtypeStruct((S_PAD, D), jnp.float32),
    )(meta, x_sorted, w1, w2x, b1, b2, w_rows)


def _w2x_body(w2_ref, o_ref):
    w = w2_ref[0]
    z = jnp.zeros_like(w)
    o_ref[...] = jnp.stack([w, z], axis=1).reshape(1, 2 * F, D)


def _tc_w2x(down_proj):
    return pl.pallas_call(
        _w2x_body,
        grid=(E,),
        in_specs=[pl.BlockSpec((1, F, D), lambda i: (i, 0, 0))],
        out_specs=pl.BlockSpec((1, 2 * F, D), lambda i: (i, 0, 0)),
        out_shape=jax.ShapeDtypeStruct((E, 2 * F, D), jnp.float32),
    )(down_proj)


ADD_BLK = 512


def _add_body(a_ref, b_ref, o_ref):
    o_ref[...] = a_ref[...] + b_ref[...]


def _pair_add(ab):
    return pl.pallas_call(
        _add_body,
        grid=(T // ADD_BLK,),
        in_specs=[
            pl.BlockSpec((ADD_BLK, D), lambda i: (i, 0)),
            pl.BlockSpec((ADD_BLK, D), lambda i: (i + T // ADD_BLK, 0)),
        ],
        out_specs=pl.BlockSpec((ADD_BLK, D), lambda i: (i, 0)),
        out_shape=jax.ShapeDtypeStruct((T, D), jnp.float32),
    )(ab, ab)


def kernel(hidden_states, router_weight, router_bias, gate_up_proj,
           gate_up_proj_bias, down_proj, down_proj_bias):
    batch = hidden_states.shape[0]
    x = hidden_states.reshape(T, D)

    logits = x @ router_weight.T + router_bias
    top_vals, top_idx = jax.lax.top_k(logits, TOP_K)
    rw = jax.nn.softmax(top_vals, axis=-1)

    e_flat = top_idx.reshape(-1).astype(jnp.int32)
    onehot = (e_flat[:, None] == jnp.arange(E, dtype=jnp.int32)[None, :])
    csum = jnp.cumsum(onehot.astype(jnp.int32), axis=0)
    rank = jnp.take_along_axis(csum, e_flat[:, None], axis=1)[:, 0] - 1
    counts = csum[-1]
    nblk = (counts + BLK - 1) // BLK
    blk_end = jnp.cumsum(nblk)
    blk_start = blk_end - nblk
    slot = blk_start[e_flat] * BLK + rank

    total_blk = blk_end[-1]
    bids = jnp.arange(NB, dtype=jnp.int32)
    bexp = jnp.sum(bids[:, None] >= blk_end[None, :], axis=1).astype(jnp.int32)
    last_e = jnp.max(jnp.where(counts > 0, jnp.arange(E, dtype=jnp.int32), -1))
    active = (bids < total_blk).astype(jnp.int32)
    bexp = jnp.where(active == 1, bexp, last_e)
    meta = jnp.concatenate([bexp, active])

    order = jnp.argsort(e_flat, stable=True).astype(jnp.int32)
    count_off = jnp.cumsum(counts) - counts
    sids = jnp.arange(S_PAD, dtype=jnp.int32)
    e_s = bexp[sids // BLK]
    r_s = sids - blk_start[e_s] * BLK
    valid_s = r_s < counts[e_s]
    p_s = jnp.clip(count_off[e_s] + r_s, 0, 2 * T - 1)
    src_token = jnp.where(valid_s, order[p_s] // TOP_K,
                          sids % T).astype(jnp.int32)
    w_flat = rw.reshape(-1)
    w_rows = jnp.where(valid_s, w_flat[order[p_s]], 0.0)
    w_rows = w_rows.reshape(NB, BLK, 1).astype(jnp.float32)
    pair_slots = jnp.concatenate([slot[0::2], slot[1::2]])

    w2x = _tc_w2x(down_proj)
    b1 = gate_up_proj_bias.reshape(E, 1, 2 * F)
    b2 = down_proj_bias.reshape(E, 1, D)

    sc_gather, sc_pair_gather = _sc_kernels()
    x_sorted = sc_gather(x, src_token)
    out_sorted = _grouped_ffn(meta, x_sorted, gate_up_proj, w2x, b1, b2,
                              w_rows)
    ab = sc_pair_gather(out_sorted, pair_slots)
    out = _pair_add(ab)
    return out.reshape(batch, T, D)
